# Initial kernel scaffold; baseline (speedup 1.0000x reference)
import jax
import jax.numpy as jnp
from jax.experimental import pallas as pl


def kernel(user, recipe, user_table, recipe_table, W1, b1, W2, b2, W3, b3):
    raise NotImplementedError("write your pallas kernel here")


# speedup vs baseline: 2.2611x; 2.2611x over previous
import functools

import jax
import jax.numpy as jnp
from jax import lax
from jax.experimental import pallas as pl
from jax.experimental.pallas import tpu as pltpu
from jax.experimental.pallas import tpu_sc as plsc

B = 16384
D = 128
NW = 32
BPW = B // NW
CHUNK = 128
NCH = BPW // CHUNK
H = 128
BLK = 512


def _sc_gather_body(uidx_hbm, ridx_hbm, utab_hbm, rtab_hbm, uout_hbm, rout_hbm,
                    idx_v, rows_v, sem):
    wid = lax.axis_index("s") * 2 + lax.axis_index("c")
    base = wid * BPW
    row0 = wid * NCH
    for idx_hbm, tab_hbm, out_hbm in ((uidx_hbm, utab_hbm, uout_hbm),
                                      (ridx_hbm, rtab_hbm, rout_hbm)):
        pltpu.sync_copy(idx_hbm.at[pl.ds(row0, NCH)], idx_v)
        copies = [
            pltpu.async_copy(tab_hbm.at[idx_v.at[ch]],
                             rows_v.at[pl.ds(ch * CHUNK, CHUNK)], sem)
            for ch in range(NCH)
        ]
        for c in copies:
            c.wait()
        pltpu.sync_copy(rows_v, out_hbm.at[pl.ds(base, BPW)])


@functools.cache
def _sc_gather():
    return pl.kernel(
        _sc_gather_body,
        out_type=[jax.ShapeDtypeStruct((B, D), jnp.float32),
                  jax.ShapeDtypeStruct((B, D), jnp.float32)],
        mesh=plsc.VectorSubcoreMesh(core_axis_name="c", subcore_axis_name="s"),
        scratch_types=[
            pltpu.VMEM((NCH, CHUNK), jnp.int32),
            pltpu.VMEM((BPW, D), jnp.float32),
            pltpu.SemaphoreType.DMA,
        ],
    )


def _mlp_body(re_ref, ue_ref, w1r_ref, w1u_ref, b1_ref, w2_ref, b2_ref,
              w3_ref, b3_ref, out_ref):
    r1 = jnp.dot(re_ref[...], w1r_ref[...], preferred_element_type=jnp.float32)
    r1 += jnp.dot(ue_ref[...], w1u_ref[...], preferred_element_type=jnp.float32)
    r1 = jnp.maximum(r1 + b1_ref[...], 0.0)
    r2 = jnp.dot(r1, w2_ref[...], preferred_element_type=jnp.float32)
    r2 = jnp.maximum(r2 + b2_ref[...], 0.0)
    s = jnp.sum(r2 * w3_ref[...], axis=1, keepdims=True) + b3_ref[...]
    out_ref[...] = s


def _mlp(re, ue, w1r, w1u, b1p, w2p, b2p, w3p, b3p):
    full = lambda shape: pl.BlockSpec(shape, lambda i: (0, 0))
    out = pl.pallas_call(
        _mlp_body,
        grid=(B // BLK,),
        in_specs=[
            pl.BlockSpec((BLK, D), lambda i: (i, 0)),
            pl.BlockSpec((BLK, D), lambda i: (i, 0)),
            full((D, H)), full((D, H)), full((1, H)),
            full((H, H)), full((1, H)),
            full((1, H)), full((1, 1)),
        ],
        out_specs=pl.BlockSpec((BLK, 1), lambda i: (i, 0)),
        out_shape=jax.ShapeDtypeStruct((B, 1), jnp.float32),
    )(re, ue, w1r, w1u, b1p, w2p, b2p, w3p, b3p)
    return out.reshape(B)


def kernel(user, recipe, user_table, recipe_table, W1, b1, W2, b2, W3, b3):
    uidx = user.astype(jnp.int32).reshape(NW * NCH, CHUNK)
    ridx = recipe.astype(jnp.int32).reshape(NW * NCH, CHUNK)
    ue, re = _sc_gather()(uidx, ridx, user_table, recipe_table)

    h1 = W1.shape[1]
    h2 = W2.shape[1]
    w1p = jnp.pad(W1, ((0, 0), (0, H - h1)))
    w1r, w1u = w1p[:D], w1p[D:]
    b1p = jnp.pad(b1, (0, H - h1)).reshape(1, H)
    w2p = jnp.pad(W2, ((0, H - h1), (0, H - h2)))
    b2p = jnp.pad(b2, (0, H - h2)).reshape(1, H)
    w3p = jnp.pad(W3[:, 0], (0, H - h2)).reshape(1, H)
    b3p = b3.reshape(1, 1)
    return _mlp(re, ue, w1r, w1u, b1p, w2p, b2p, w3p, b3p)


# speedup vs baseline: 2.6731x; 1.1822x over previous
import functools

import jax
import jax.numpy as jnp
from jax import lax
from jax.experimental import pallas as pl
from jax.experimental.pallas import tpu as pltpu
from jax.experimental.pallas import tpu_sc as plsc

B = 16384
D = 128
NW = 32
BPW = B // NW
CHUNK = 128
NCH = BPW // CHUNK
H = 128
BLK = 1024


def _sc_gather_body(uidx_hbm, ridx_hbm, utab_hbm, rtab_hbm, uout_hbm, rout_hbm,
                    idx_v, rows_v, sem):
    wid = lax.axis_index("s") * 2 + lax.axis_index("c")
    base = wid * BPW
    row0 = wid * NCH
    for idx_hbm, tab_hbm, out_hbm in ((uidx_hbm, utab_hbm, uout_hbm),
                                      (ridx_hbm, rtab_hbm, rout_hbm)):
        pltpu.sync_copy(idx_hbm.at[pl.ds(row0, NCH)], idx_v)
        copies = [
            pltpu.async_copy(tab_hbm.at[idx_v.at[ch]],
                             rows_v.at[pl.ds(ch * CHUNK, CHUNK)], sem)
            for ch in range(NCH)
        ]
        for c in copies:
            c.wait()
        pltpu.sync_copy(rows_v, out_hbm.at[pl.ds(base, BPW)])


@functools.cache
def _sc_gather():
    return pl.kernel(
        _sc_gather_body,
        out_type=[jax.ShapeDtypeStruct((B, D), jnp.float32),
                  jax.ShapeDtypeStruct((B, D), jnp.float32)],
        mesh=plsc.VectorSubcoreMesh(core_axis_name="c", subcore_axis_name="s"),
        scratch_types=[
            pltpu.VMEM((NCH, CHUNK), jnp.int32),
            pltpu.VMEM((BPW, D), jnp.float32),
            pltpu.SemaphoreType.DMA,
        ],
    )


def _mlp_body(re_ref, ue_ref, w1r_ref, w1u_ref, b1_ref, w2_ref, b2_ref,
              w3_ref, b3_ref, out_ref):
    reb = re_ref[...].astype(jnp.bfloat16)
    ueb = ue_ref[...].astype(jnp.bfloat16)
    r1 = jnp.dot(reb, w1r_ref[...], preferred_element_type=jnp.float32)
    r1 += jnp.dot(ueb, w1u_ref[...], preferred_element_type=jnp.float32)
    r1 = jnp.maximum(r1 + b1_ref[...], 0.0).astype(jnp.bfloat16)
    r2 = jnp.dot(r1, w2_ref[...], preferred_element_type=jnp.float32)
    r2 = jnp.maximum(r2 + b2_ref[...], 0.0)
    out_ref[...] = jnp.sum(r2 * w3_ref[...], axis=1, keepdims=True) + b3_ref[...]


def _mlp(re, ue, w1r, w1u, b1p, w2p, b2p, w3p, b3p):
    full = lambda shape: pl.BlockSpec(shape, lambda i: (0, 0))
    return pl.pallas_call(
        _mlp_body,
        grid=(B // BLK,),
        in_specs=[
            pl.BlockSpec((BLK, D), lambda i: (i, 0)),
            pl.BlockSpec((BLK, D), lambda i: (i, 0)),
            full((D, H)), full((D, H)), full((1, H)),
            full((H, H)), full((1, H)),
            full((1, H)), full((1, 1)),
        ],
        out_specs=pl.BlockSpec((BLK, 1), lambda i: (i, 0)),
        out_shape=jax.ShapeDtypeStruct((B, 1), jnp.float32),
    )(re, ue, w1r, w1u, b1p, w2p, b2p, w3p, b3p).reshape(B)


def kernel(user, recipe, user_table, recipe_table, W1, b1, W2, b2, W3, b3):
    uidx = user.astype(jnp.int32).reshape(NW * NCH, CHUNK)
    ridx = recipe.astype(jnp.int32).reshape(NW * NCH, CHUNK)
    ue, re = _sc_gather()(uidx, ridx, user_table, recipe_table)

    h1 = W1.shape[1]
    h2 = W2.shape[1]
    w1p = jnp.pad(W1, ((0, 0), (0, H - h1))).astype(jnp.bfloat16)
    w1r, w1u = w1p[:D], w1p[D:]
    b1p = jnp.pad(b1, (0, H - h1)).reshape(1, H)
    w2p = jnp.pad(W2, ((0, H - h1), (0, H - h2))).astype(jnp.bfloat16)
    b2p = jnp.pad(b2, (0, H - h2)).reshape(1, H)
    w3p = jnp.pad(W3[:, 0], (0, H - h2)).reshape(1, H)
    b3p = b3.reshape(1, 1)
    return _mlp(re, ue, w1r, w1u, b1p, w2p, b2p, w3p, b3p)
